# baseline (speedup 1.0000x reference)
import functools

import numpy as np
import jax
import jax.numpy as jnp
from jax import lax
from jax.experimental import pallas as pl
from jax.experimental.pallas import tpu as pltpu
from jax.experimental.pallas import tpu_sc as plsc


def _sc_row_gather(flat, rows, n_workers=32, num_cores=2):
    n_rows = len(rows)
    d = flat.shape[1]
    n_chunks = 2 * n_rows
    assert n_chunks % n_workers == 0
    per_w = n_chunks // n_workers
    ch = d // 2

    mesh = plsc.VectorSubcoreMesh(core_axis_name="c", subcore_axis_name="s")

    @functools.partial(
        pl.kernel,
        mesh=mesh,
        out_type=jax.ShapeDtypeStruct((n_rows, d), flat.dtype),
        scratch_types=[
            pltpu.VMEM((per_w * ch,), flat.dtype),
            pltpu.SemaphoreType.DMA,
        ],
    )
    def k(flat_ref, out_ref, buf, sem):
        w = lax.axis_index("s") * num_cores + lax.axis_index("c")
        for wo in range(n_workers):

            @pl.when(w == wo)
            def _():
                reads = []
                for t in range(per_w):
                    q = wo * per_w + t
                    j, h = q // 2, q % 2
                    reads.append(
                        pltpu.async_copy(
                            flat_ref.at[rows[j], pl.ds(h * ch, ch)],
                            buf.at[pl.ds(t * ch, ch)],
                            sem,
                        )
                    )
                for r in reads:
                    r.wait()
                writes = []
                for t in range(per_w):
                    q = wo * per_w + t
                    j, h = q // 2, q % 2
                    writes.append(
                        pltpu.async_copy(
                            buf.at[pl.ds(t * ch, ch)],
                            out_ref.at[j, pl.ds(h * ch, ch)],
                            sem,
                        )
                    )
                for wr in writes:
                    wr.wait()

    return k(flat)


def kernel(frames):
    c, t, h, w = frames.shape
    t_out = t // 4
    i = np.arange(t_out - 1, dtype=np.float32)
    frac = i / np.float32(t_out - 1)
    vals = np.float32(0.0) * (np.float32(1.0) - frac) + np.float32(t - 1) * frac
    idx = np.concatenate([vals, [np.float32(t - 1)]]).astype(np.int32)
    rows = tuple(int(ci) * t + int(ti) for ci in range(c) for ti in idx)
    flat = frames.reshape(c * t, h * w)
    slow = _sc_row_gather(flat, rows)
    return slow.reshape(c, t_out, h, w), frames


# speedup vs baseline: 2.0104x; 2.0104x over previous
import functools

import numpy as np
import jax
import jax.numpy as jnp
from jax import lax
from jax.experimental import pallas as pl
from jax.experimental.pallas import tpu as pltpu
from jax.experimental.pallas import tpu_sc as plsc

_N_WORKERS = 32
_N_CORES = 2


def _linspace_floor_idx(t, t_out):
    i = np.arange(t_out - 1, dtype=np.float32)
    frac = i / np.float32(t_out - 1)
    vals = np.float32(0.0) * (np.float32(1.0) - frac) + np.float32(t - 1) * frac
    return np.concatenate([vals, [np.float32(t - 1)]]).astype(np.int32)


def _sc_frame_gather(frames, pairs, t_out):
    c, t, h, w = frames.shape
    n_chunks = 2 * len(pairs)
    assert n_chunks % _N_WORKERS == 0
    per_w = n_chunks // _N_WORKERS
    hh = h // 2

    mesh = plsc.VectorSubcoreMesh(core_axis_name="c", subcore_axis_name="s")

    @functools.partial(
        pl.kernel,
        mesh=mesh,
        out_type=jax.ShapeDtypeStruct((c, t_out, h, w), frames.dtype),
        scratch_types=[
            pltpu.VMEM((per_w, hh, w), frames.dtype),
            pltpu.SemaphoreType.DMA,
        ],
    )
    def k(frames_ref, out_ref, buf, sem):
        wid = lax.axis_index("s") * _N_CORES + lax.axis_index("c")
        for wo in range(_N_WORKERS):

            @pl.when(wid == wo)
            def _():
                reads = []
                for k_ in range(per_w):
                    q = wo * per_w + k_
                    j, half = q // 2, q % 2
                    ci, ti = pairs[j]
                    reads.append(
                        pltpu.async_copy(
                            frames_ref.at[ci, ti, pl.ds(half * hh, hh), :],
                            buf.at[k_],
                            sem,
                        )
                    )
                for r in reads:
                    r.wait()
                writes = []
                for k_ in range(per_w):
                    q = wo * per_w + k_
                    j, half = q // 2, q % 2
                    writes.append(
                        pltpu.async_copy(
                            buf.at[k_],
                            out_ref.at[j // t_out, j % t_out, pl.ds(half * hh, hh), :],
                            sem,
                        )
                    )
                for wr in writes:
                    wr.wait()

    return k(frames)


def _tc_copy(frames):
    c, t, h, w = frames.shape
    tb = 16

    def body(src, dst):
        dst[...] = src[...]

    return pl.pallas_call(
        body,
        grid=(c, t // tb),
        in_specs=[pl.BlockSpec((1, tb, h, w), lambda ci, ti: (ci, ti, 0, 0))],
        out_specs=pl.BlockSpec((1, tb, h, w), lambda ci, ti: (ci, ti, 0, 0)),
        out_shape=jax.ShapeDtypeStruct(frames.shape, frames.dtype),
    )(frames)


def kernel(frames):
    c, t, h, w = frames.shape
    t_out = t // 4
    idx = _linspace_floor_idx(t, t_out)
    pairs = tuple((ci, int(ti)) for ci in range(c) for ti in idx)
    slow = _sc_frame_gather(frames, pairs, t_out)
    fast = _tc_copy(frames)
    return slow, fast


# speedup vs baseline: 2.0618x; 1.0256x over previous
import functools

import numpy as np
import jax
import jax.numpy as jnp
from jax import lax
from jax.experimental import pallas as pl
from jax.experimental.pallas import tpu as pltpu
from jax.experimental.pallas import tpu_sc as plsc

_N_WORKERS = 32
_N_CORES = 2


def _linspace_floor_idx(t, t_out):
    i = np.arange(t_out - 1, dtype=np.float32)
    frac = i / np.float32(t_out - 1)
    vals = np.float32(0.0) * (np.float32(1.0) - frac) + np.float32(t - 1) * frac
    return np.concatenate([vals, [np.float32(t - 1)]]).astype(np.int32)


def _sc_frame_gather(frames, pairs, t_out):
    c, t, h, w = frames.shape
    n_planes = len(pairs)

    assign = []
    p = 0
    for wo in range(_N_WORKERS):
        take = 2 if wo < n_planes - _N_WORKERS else 1
        assign.append(tuple(range(p, p + take)))
        p += take
    assert p == n_planes

    mesh = plsc.VectorSubcoreMesh(core_axis_name="c", subcore_axis_name="s")

    @functools.partial(
        pl.kernel,
        mesh=mesh,
        out_type=jax.ShapeDtypeStruct((c, t_out, h, w), frames.dtype),
        scratch_types=[
            pltpu.VMEM((h, w), frames.dtype),
            pltpu.SemaphoreType.DMA,
        ],
    )
    def k(frames_ref, out_ref, buf, sem):
        wid = lax.axis_index("s") * _N_CORES + lax.axis_index("c")
        for wo in range(_N_WORKERS):

            @pl.when(wid == wo)
            def _():
                for j in assign[wo]:
                    ci, ti = pairs[j]
                    pltpu.async_copy(frames_ref.at[ci, ti], buf, sem).wait()
                    pltpu.async_copy(
                        buf, out_ref.at[j // t_out, j % t_out], sem
                    ).wait()

    return k(frames)


def _tc_copy(frames):
    c, t, h, w = frames.shape
    tb = 32

    def body(src, dst):
        dst[...] = src[...]

    return pl.pallas_call(
        body,
        grid=(c, t // tb),
        in_specs=[pl.BlockSpec((1, tb, h, w), lambda ci, ti: (ci, ti, 0, 0))],
        out_specs=pl.BlockSpec((1, tb, h, w), lambda ci, ti: (ci, ti, 0, 0)),
        out_shape=jax.ShapeDtypeStruct(frames.shape, frames.dtype),
    )(frames)


def kernel(frames):
    c, t, h, w = frames.shape
    t_out = t // 4
    idx = _linspace_floor_idx(t, t_out)
    pairs = tuple((ci, int(ti)) for ci in range(c) for ti in idx)
    slow = _sc_frame_gather(frames, pairs, t_out)
    fast = _tc_copy(frames)
    return slow, fast
